# baseline (speedup 1.0000x reference)
import functools

import jax
import jax.numpy as jnp
from jax import lax
from jax.experimental import pallas as pl
from jax.experimental.pallas import tpu as pltpu
from jax.experimental.pallas import tpu_sc as plsc

N = 10000
NPAD = 10240
E = 320000
D = 128
HID = 128
HEADS = 8
EMB = 128
F = 144
NTILES = 32
NBLK = 6
BCH = 27
CK = 64
BLKE = BCH * CK
EPAD = NTILES * NBLK * BLKE
ROWS_PER_TILE = NPAD // 16


def _leaky(u):
    return jnp.maximum(u, 0.2 * u)


def _l1_prep_body(x_ref, w1_ref, asbd_ref, adbd_ref,
                  haug_ref, as_ref, ad_ref, am_ref):
    i = pl.program_id(0)
    h = jnp.dot(x_ref[...], w1_ref[...], preferred_element_type=jnp.float32)
    as_blk = jnp.dot(h, asbd_ref[...], preferred_element_type=jnp.float32)
    as_ref[...] = as_blk
    ad_ref[...] = jnp.dot(h, adbd_ref[...], preferred_element_type=jnp.float32)
    pad = jnp.concatenate(
        [jnp.ones((128, 1), jnp.float32), jnp.zeros((128, 15), jnp.float32)], axis=1)
    for hd in range(HEADS):
        haug_ref[hd, :, 0:128] = h[:, hd * 128:(hd + 1) * 128]
        haug_ref[hd, :, 128:144] = pad

    ridx = i * 128 + lax.broadcasted_iota(jnp.int32, (128, HEADS), 0)
    masked = jnp.where(ridx < N, as_blk, -3e38)
    bmax = jnp.max(masked, axis=0, keepdims=True)
    cur = jnp.broadcast_to(bmax, (16, HEADS))

    @pl.when(i == 0)
    def _():
        am_ref[...] = cur

    @pl.when(i > 0)
    def _():
        am_ref[...] = jnp.maximum(am_ref[...], cur)


def _l1_prep(xpad, W1, asbd, adbd):
    return pl.pallas_call(
        _l1_prep_body,
        grid=(NPAD // 128,),
        in_specs=[
            pl.BlockSpec((128, D), lambda i: (i, 0)),
            pl.BlockSpec((D, HEADS * HID), lambda i: (0, 0)),
            pl.BlockSpec((HEADS * HID, HEADS), lambda i: (0, 0)),
            pl.BlockSpec((HEADS * HID, HEADS), lambda i: (0, 0)),
        ],
        out_specs=[
            pl.BlockSpec((HEADS, 128, F), lambda i: (0, i, 0)),
            pl.BlockSpec((128, HEADS), lambda i: (i, 0)),
            pl.BlockSpec((128, HEADS), lambda i: (i, 0)),
            pl.BlockSpec((16, HEADS), lambda i: (0, 0)),
        ],
        out_shape=[
            jax.ShapeDtypeStruct((HEADS, NPAD, F), jnp.float32),
            jax.ShapeDtypeStruct((NPAD, HEADS), jnp.float32),
            jax.ShapeDtypeStruct((NPAD, HEADS), jnp.float32),
            jax.ShapeDtypeStruct((16, HEADS), jnp.float32),
        ],
    )(xpad, W1, asbd, adbd)


def _sc_wk_body(nheads, ast, adt, amx, srcf, dstf, wout_ref, gout_ref,
                as_v, ad_v, am_v, sblk_v, dblk_v, wv, gv):
    c = lax.axis_index("c")
    s = lax.axis_index("s")
    wid = c * 16 + s

    def head_loop(hd, _):
        pltpu.sync_copy(ast.at[hd], as_v)
        pltpu.sync_copy(adt.at[hd], ad_v)
        pltpu.sync_copy(amx.at[hd], am_v)
        mvec = am_v[...]
        base = hd * NPAD

        def blk_loop(blk, _):
            pltpu.sync_copy(srcf.at[wid, blk], sblk_v)
            pltpu.sync_copy(dstf.at[wid, blk], dblk_v)

            for g in range(BLKE // 16):
                sl = pl.ds(g * 16, 16)
                sv = sblk_v[sl]
                dv = dblk_v[sl]
                asg = plsc.load_gather(as_v, [sv])
                adg = plsc.load_gather(ad_v, [dv])
                u = asg + adg
                cc = _leaky(adg + mvec)
                wv[sl] = jnp.exp(_leaky(u) - cc)
                gv[sl] = sv + base
            pltpu.sync_copy(wv, wout_ref.at[hd, wid, blk])
            pltpu.sync_copy(gv, gout_ref.at[hd, wid, blk])
            return 0
        lax.fori_loop(0, NBLK, blk_loop, 0)
        return 0
    lax.fori_loop(0, nheads, head_loop, 0)


def _sc_wk(nheads, ast, adt, amx, srcf, dstf):
    mesh = plsc.VectorSubcoreMesh(core_axis_name="c", subcore_axis_name="s")
    kfn = pl.kernel(
        functools.partial(_sc_wk_body, nheads),
        out_type=[
            jax.ShapeDtypeStruct((nheads, NTILES, NBLK, BLKE), jnp.float32),
            jax.ShapeDtypeStruct((nheads, NTILES, NBLK, BLKE), jnp.int32),
        ],
        mesh=mesh,
        scratch_types=[
            pltpu.VMEM((NPAD,), jnp.float32),
            pltpu.VMEM((NPAD,), jnp.float32),
            pltpu.VMEM((16,), jnp.float32),
            pltpu.VMEM((BLKE,), jnp.int32),
            pltpu.VMEM((BLKE,), jnp.int32),
            pltpu.VMEM((BLKE,), jnp.float32),
            pltpu.VMEM((BLKE,), jnp.int32),
        ],
        compiler_params=pltpu.CompilerParams(
            needs_layout_passes=False, use_tc_tiling_on_sc=False),
    )
    return kfn(ast, adt, amx, srcf, dstf)


def _sc_agg_body(nheads, htab, g5, w4, d5, out_ref,
                 rows0, rows1, rows2, dblk_v, gblk_v, wblk_v,
                 gs0, gs1, gs2, ss0, ss1, ss2,
                 out_sp):
    c = lax.axis_index("c")
    s = lax.axis_index("s")
    wid = c * 16 + s
    rows = (rows0, rows1, rows2)
    gsem = (gs0, gs1, gs2)
    ssem = (ss0, ss1, ss2)
    zero16 = jnp.zeros((16,), jnp.float32)

    def head_loop(hd, _):
        def zr(i, _):
            for j in range(F // 16):
                rows2[i, pl.ds(j * 16, 16)] = zero16
            return 0
        lax.fori_loop(0, CK, zr, 0)

        def zo(i, _):
            pltpu.sync_copy(rows2,
                            out_sp.at[pl.ds(s * ROWS_PER_TILE + i * CK, CK)])
            return 0
        lax.fori_loop(0, ROWS_PER_TILE // CK, zo, 0)
        plsc.subcore_barrier()

        def blk_loop(blk, _):
            pltpu.sync_copy(d5.at[wid, blk], dblk_v)
            pltpu.sync_copy(g5.at[hd, wid, blk], gblk_v)
            pltpu.sync_copy(w4.at[hd, wid, blk], wblk_v)
            pltpu.async_copy(htab.at[gblk_v.at[0]], rows0, gs0)
            pltpu.async_copy(htab.at[gblk_v.at[1]], rows1, gs1)

            def step(ch, b, wait_prev, prefetch):
                rb = rows[b]
                pltpu.make_async_copy(htab.at[gblk_v.at[ch]], rb,
                                      gsem[b]).wait()

                for k in range(CK):
                    if k % 16 == 0:
                        w16 = wblk_v[pl.ds(ch * CK + k, 16)]
                    wk = jnp.broadcast_to(w16[k % 16], (16,))
                    for j in range(F // 16):
                        fsl = pl.ds(j * 16, 16)
                        rb[k, fsl] = rb[k, fsl] * wk

                bn = (b + 2) % 3

                def _drain_prefetch():
                    if wait_prev is None:
                        pltpu.make_async_copy(
                            rows[bn], out_sp.at[dblk_v.at[ch - 1]],
                            ssem[bn]).wait()
                    else:
                        @pl.when(wait_prev)
                        def _():
                            pltpu.make_async_copy(
                                rows[bn], out_sp.at[dblk_v.at[ch - 1]],
                                ssem[bn]).wait()
                    pltpu.async_copy(htab.at[gblk_v.at[ch + 2]], rows[bn],
                                     gsem[bn])

                if prefetch is None:
                    _drain_prefetch()
                else:
                    pl.when(prefetch)(_drain_prefetch)

                pltpu.async_copy(rb, out_sp.at[dblk_v.at[ch]], ssem[b],
                                 add=True)

            def triple(t, _):
                step(3 * t, 0, t >= 1, None)
                step(3 * t + 1, 1, None, t < 8)
                step(3 * t + 2, 2, None, t < 8)
                return 0
            lax.fori_loop(0, BCH // 3, triple, 0)
            for b, chl in ((0, BCH - 3), (1, BCH - 2), (2, BCH - 1)):
                pltpu.make_async_copy(rows[b], out_sp.at[dblk_v.at[chl]],
                                      ssem[b]).wait()
            return 0
        lax.fori_loop(0, NBLK, blk_loop, 0)
        plsc.subcore_barrier()

        def wo(i, _):
            r = s * ROWS_PER_TILE + i * 64
            pltpu.sync_copy(out_sp.at[pl.ds(r, 64)],
                            out_ref.at[c, hd, pl.ds(r, 64)])
            return 0
        lax.fori_loop(0, ROWS_PER_TILE // 64, wo, 0)
        plsc.subcore_barrier()
        return 0
    lax.fori_loop(0, nheads, head_loop, 0)


def _sc_aggregate(nheads, htab, ast, adt, amx, srcf, dstf, d5):
    w4, g4 = _sc_wk(nheads, ast, adt, amx, srcf, dstf)
    g5 = g4.reshape(nheads, NTILES, NBLK, BCH, CK)
    mesh = plsc.VectorSubcoreMesh(core_axis_name="c", subcore_axis_name="s")
    kfn = pl.kernel(
        functools.partial(_sc_agg_body, nheads),
        out_type=jax.ShapeDtypeStruct((2, nheads, NPAD, F), jnp.float32),
        mesh=mesh,
        scratch_types=[
            pltpu.VMEM((CK, F), jnp.float32),
            pltpu.VMEM((CK, F), jnp.float32),
            pltpu.VMEM((CK, F), jnp.float32),
            pltpu.VMEM((BCH, CK), jnp.int32),
            pltpu.VMEM((BCH, CK), jnp.int32),
            pltpu.VMEM((BLKE,), jnp.float32),
            pltpu.SemaphoreType.DMA,
            pltpu.SemaphoreType.DMA,
            pltpu.SemaphoreType.DMA,
            pltpu.SemaphoreType.DMA,
            pltpu.SemaphoreType.DMA,
            pltpu.SemaphoreType.DMA,
            pltpu.VMEM_SHARED((NPAD, F), jnp.float32),
        ],
        compiler_params=pltpu.CompilerParams(
            needs_layout_passes=False, use_tc_tiling_on_sc=False),
    )
    return kfn(htab, g5, w4, d5)


def _l2_prep_body(p_ref, b1_ref, w2_ref, as2v_ref, ad2v_ref,
                  haug_ref, as2_ref, ad2_ref, am_ref):
    i = pl.program_id(0)
    h2 = jnp.zeros((128, EMB), jnp.float32)
    for hd in range(HEADS):
        num = p_ref[0, hd, :, 0:128] + p_ref[1, hd, :, 0:128]
        den = p_ref[0, hd, :, 128:129] + p_ref[1, hd, :, 128:129]
        g = num / (den + 1e-16) + b1_ref[hd:hd + 1, :]
        e = jnp.where(g > 0, g, jnp.exp(g) - 1.0)
        h2 = h2 + jnp.dot(e, w2_ref[hd * 128:(hd + 1) * 128, :],
                          preferred_element_type=jnp.float32)
    pad = jnp.concatenate(
        [jnp.ones((128, 1), jnp.float32), jnp.zeros((128, 15), jnp.float32)], axis=1)
    haug_ref[:, 0:128] = h2
    haug_ref[:, 128:144] = pad
    as2_blk = jnp.sum(h2 * as2v_ref[0:1, :], axis=1, keepdims=True)
    as2_ref[0, 0, :] = as2_blk[:, 0]
    ad2_ref[0, 0, :] = jnp.sum(h2 * ad2v_ref[0:1, :], axis=1)

    ridx = i * 128 + lax.broadcasted_iota(jnp.int32, (128, 1), 0)
    masked = jnp.where(ridx < N, as2_blk, -3e38)
    cur = jnp.broadcast_to(jnp.max(masked, axis=0, keepdims=True), (16, 1))

    @pl.when(i == 0)
    def _():
        am_ref[...] = cur

    @pl.when(i > 0)
    def _():
        am_ref[...] = jnp.maximum(am_ref[...], cur)


def _l2_prep(part1, b1_2d, W2, a_src2, a_dst2):
    return pl.pallas_call(
        _l2_prep_body,
        grid=(NPAD // 128,),
        in_specs=[
            pl.BlockSpec((2, HEADS, 128, F), lambda i: (0, 0, i, 0)),
            pl.BlockSpec((HEADS, HID), lambda i: (0, 0)),
            pl.BlockSpec((HEADS * HID, EMB), lambda i: (0, 0)),
            pl.BlockSpec((1, EMB), lambda i: (0, 0)),
            pl.BlockSpec((1, EMB), lambda i: (0, 0)),
        ],
        out_specs=[
            pl.BlockSpec((128, F), lambda i: (i, 0)),
            pl.BlockSpec((1, 1, 128), lambda i: (i, 0, 0)),
            pl.BlockSpec((1, 1, 128), lambda i: (i, 0, 0)),
            pl.BlockSpec((16, 1), lambda i: (0, 0)),
        ],
        out_shape=[
            jax.ShapeDtypeStruct((NPAD, F), jnp.float32),
            jax.ShapeDtypeStruct((NPAD // 128, 1, 128), jnp.float32),
            jax.ShapeDtypeStruct((NPAD // 128, 1, 128), jnp.float32),
            jax.ShapeDtypeStruct((16, 1), jnp.float32),
        ],
    )(part1, b1_2d, W2, a_src2, a_dst2)


def _final_body(p_ref, b2_ref, out_ref):
    num = p_ref[0, :, 0:128] + p_ref[1, :, 0:128]
    den = p_ref[0, :, 128:129] + p_ref[1, :, 128:129]
    z = num / (den + 1e-16) + b2_ref[0:1, :]
    m = jnp.max(z, axis=1, keepdims=True)
    zs = z - m
    out_ref[...] = zs - jnp.log(jnp.sum(jnp.exp(zs), axis=1, keepdims=True))


def _final(part2, b2_2d):
    return pl.pallas_call(
        _final_body,
        grid=(79,),
        in_specs=[
            pl.BlockSpec((2, 128, F), lambda i: (0, i, 0)),
            pl.BlockSpec((1, EMB), lambda i: (0, 0)),
        ],
        out_specs=pl.BlockSpec((128, EMB), lambda i: (i, 0)),
        out_shape=jax.ShapeDtypeStruct((N, EMB), jnp.float32),
    )(part2, b2_2d)


def kernel(x, edge_index, W1, a_src1, a_dst1, b1, W2, a_src2, a_dst2, b2):
    ei = edge_index.astype(jnp.int32)
    loop = jnp.arange(N, dtype=jnp.int32)
    padn = EPAD - (E + N)
    src = jnp.concatenate([ei[0], loop, jnp.zeros((padn,), jnp.int32)])
    dst = jnp.concatenate([ei[1], loop, jnp.full((padn,), N + 10, jnp.int32)])
    srcf = src.reshape(NTILES, NBLK, BLKE)
    dstf = dst.reshape(NTILES, NBLK, BLKE)
    d5 = dst.reshape(NTILES, NBLK, BCH, CK)

    xpad = jnp.pad(x, ((0, NPAD - N), (0, 0)))

    eye = jnp.eye(HEADS, dtype=jnp.float32)
    asbd = (eye[:, None, :] * a_src1[:, :, None]).reshape(HEADS * HID, HEADS)
    adbd = (eye[:, None, :] * a_dst1[:, :, None]).reshape(HEADS * HID, HEADS)

    haug1, as1, ad1, am1 = _l1_prep(xpad, W1, asbd, adbd)

    htab1 = haug1.reshape(HEADS * NPAD, F)
    ast1 = as1.T.reshape(HEADS, NPAD)
    adt1 = ad1.T.reshape(HEADS, NPAD)
    amx1 = am1.T.reshape(HEADS, 16)
    part1 = _sc_aggregate(HEADS, htab1, ast1, adt1, amx1, srcf, dstf, d5)

    b1_2d = b1.reshape(HEADS, HID)
    haug2, as2, ad2, am2 = _l2_prep(part1, b1_2d, W2, a_src2, a_dst2)

    ast2 = as2.reshape(1, NPAD)
    adt2 = ad2.reshape(1, NPAD)
    amx2 = am2.T.reshape(1, 16)
    part2 = _sc_aggregate(1, haug2, ast2, adt2, amx2, srcf, dstf, d5)

    return _final(part2.reshape(2, NPAD, F), b2.reshape(1, EMB))


# speedup vs baseline: 1.3559x; 1.3559x over previous
import functools

import jax
import jax.numpy as jnp
from jax import lax
from jax.experimental import pallas as pl
from jax.experimental.pallas import tpu as pltpu
from jax.experimental.pallas import tpu_sc as plsc

N = 10000
NPAD = 10240
E = 320000
D = 128
HID = 128
HEADS = 8
EMB = 128
F = 144
NTILES = 32
NBLK = 8
BCH = 27
CK = 48
BLKE = BCH * CK
EPAD = NTILES * NBLK * BLKE
ROWS_PER_TILE = NPAD // 16
ZCH = 40


def _leaky(u):
    return jnp.maximum(u, 0.2 * u)


def _l1_prep_body(x_ref, w1_ref, asbd_ref, adbd_ref,
                  haug_ref, as_ref, ad_ref, am_ref):
    i = pl.program_id(0)
    h = jnp.dot(x_ref[...], w1_ref[...], preferred_element_type=jnp.float32)
    as_blk = jnp.dot(h, asbd_ref[...], preferred_element_type=jnp.float32)
    as_ref[...] = as_blk
    ad_ref[...] = jnp.dot(h, adbd_ref[...], preferred_element_type=jnp.float32)
    for hd in range(HEADS):
        haug_ref[hd, :, :] = h[:, hd * 128:(hd + 1) * 128].astype(jnp.bfloat16)

    ridx = i * 128 + lax.broadcasted_iota(jnp.int32, (128, HEADS), 0)
    masked = jnp.where(ridx < N, as_blk, -3e38)
    bmax = jnp.max(masked, axis=0, keepdims=True)
    cur = jnp.broadcast_to(bmax, (16, HEADS))

    @pl.when(i == 0)
    def _():
        am_ref[...] = cur

    @pl.when(i > 0)
    def _():
        am_ref[...] = jnp.maximum(am_ref[...], cur)


def _l1_prep(xpad, W1, asbd, adbd):
    return pl.pallas_call(
        _l1_prep_body,
        grid=(NPAD // 128,),
        in_specs=[
            pl.BlockSpec((128, D), lambda i: (i, 0)),
            pl.BlockSpec((D, HEADS * HID), lambda i: (0, 0)),
            pl.BlockSpec((HEADS * HID, HEADS), lambda i: (0, 0)),
            pl.BlockSpec((HEADS * HID, HEADS), lambda i: (0, 0)),
        ],
        out_specs=[
            pl.BlockSpec((HEADS, 128, 128), lambda i: (0, i, 0)),
            pl.BlockSpec((128, HEADS), lambda i: (i, 0)),
            pl.BlockSpec((128, HEADS), lambda i: (i, 0)),
            pl.BlockSpec((16, HEADS), lambda i: (0, 0)),
        ],
        out_shape=[
            jax.ShapeDtypeStruct((HEADS, NPAD, 128), jnp.bfloat16),
            jax.ShapeDtypeStruct((NPAD, HEADS), jnp.float32),
            jax.ShapeDtypeStruct((NPAD, HEADS), jnp.float32),
            jax.ShapeDtypeStruct((16, HEADS), jnp.float32),
        ],
    )(xpad, W1, asbd, adbd)


def _sc_wk_body(nheads, ast, adt, amx, srcf, dstf, wout_ref, gout_ref,
                as_v, ad_v, am_v, sblk_v, dblk_v, wv, gv):
    c = lax.axis_index("c")
    s = lax.axis_index("s")
    wid = c * 16 + s

    def head_loop(hd, _):
        pltpu.sync_copy(ast.at[hd], as_v)
        pltpu.sync_copy(adt.at[hd], ad_v)
        pltpu.sync_copy(amx.at[hd], am_v)
        mvec = am_v[...]
        base = hd * NPAD

        def blk_loop(blk, _):
            pltpu.sync_copy(srcf.at[wid, blk], sblk_v)
            pltpu.sync_copy(dstf.at[wid, blk], dblk_v)

            for g in range(BLKE // 16):
                sl = pl.ds(g * 16, 16)
                sv = sblk_v[sl]
                dv = dblk_v[sl]
                asg = plsc.load_gather(as_v, [sv])
                adg = plsc.load_gather(ad_v, [dv])
                u = asg + adg
                cc = _leaky(adg + mvec)
                wv[sl] = jnp.exp(_leaky(u) - cc)
                gv[sl] = sv + base
            pltpu.sync_copy(wv, wout_ref.at[hd, wid, blk])
            pltpu.sync_copy(gv, gout_ref.at[hd, wid, blk])
            return 0
        lax.fori_loop(0, NBLK, blk_loop, 0)
        return 0
    lax.fori_loop(0, nheads, head_loop, 0)


def _sc_wk(nheads, ast, adt, amx, srcf, dstf):
    mesh = plsc.VectorSubcoreMesh(core_axis_name="c", subcore_axis_name="s")
    kfn = pl.kernel(
        functools.partial(_sc_wk_body, nheads),
        out_type=[
            jax.ShapeDtypeStruct((nheads, NTILES, NBLK, BLKE), jnp.float32),
            jax.ShapeDtypeStruct((nheads, NTILES, NBLK, BLKE), jnp.int32),
        ],
        mesh=mesh,
        scratch_types=[
            pltpu.VMEM((NPAD,), jnp.float32),
            pltpu.VMEM((NPAD,), jnp.float32),
            pltpu.VMEM((16,), jnp.float32),
            pltpu.VMEM((BLKE,), jnp.int32),
            pltpu.VMEM((BLKE,), jnp.int32),
            pltpu.VMEM((BLKE,), jnp.float32),
            pltpu.VMEM((BLKE,), jnp.int32),
        ],
        compiler_params=pltpu.CompilerParams(
            needs_layout_passes=False, use_tc_tiling_on_sc=False),
    )
    return kfn(ast, adt, amx, srcf, dstf)


def _sc_agg_body(nheads, htab, g5, w4, d5, out_ref,
                 rb0, rb1, rb2, rf0, rf1, rf2, dblk_v, gblk_v, wblk_v,
                 gs0, gs1, gs2, ss0, ss1, ss2,
                 out_sp):
    c = lax.axis_index("c")
    s = lax.axis_index("s")
    wid = c * 16 + s
    rowsb = (rb0, rb1, rb2)
    rowsf = (rf0, rf1, rf2)
    gsem = (gs0, gs1, gs2)
    ssem = (ss0, ss1, ss2)
    zero16 = jnp.zeros((16,), jnp.float32)
    onehot0 = jnp.where(lax.iota(jnp.int32, 16) == 0,
                        jnp.full((16,), 1.0, jnp.float32),
                        jnp.zeros((16,), jnp.float32))

    def head_loop(hd, _):
        def zr(i, _):
            for j in range(F // 16):
                rf0[i, pl.ds(j * 16, 16)] = zero16
            return 0
        lax.fori_loop(0, ZCH, zr, 0)

        def zo(i, _):
            pltpu.sync_copy(rf0.at[pl.ds(0, ZCH)],
                            out_sp.at[pl.ds(s * ROWS_PER_TILE + i * ZCH, ZCH)])
            return 0
        lax.fori_loop(0, ROWS_PER_TILE // ZCH, zo, 0)
        plsc.subcore_barrier()

        def blk_loop(blk, _):
            pltpu.sync_copy(d5.at[wid, blk], dblk_v)
            pltpu.sync_copy(g5.at[hd, wid, blk], gblk_v)
            pltpu.sync_copy(w4.at[hd, wid, blk], wblk_v)
            pltpu.async_copy(htab.at[gblk_v.at[0]], rb0, gs0)
            pltpu.async_copy(htab.at[gblk_v.at[1]], rb1, gs1)

            def step(ch, b, wait3, prefetch):
                rb = rowsb[b]
                rf = rowsf[b]
                pltpu.make_async_copy(htab.at[gblk_v.at[ch]], rb,
                                      gsem[b]).wait()

                bn = (b + 2) % 3

                def _prefetch():
                    pltpu.async_copy(htab.at[gblk_v.at[ch + 2]], rowsb[bn],
                                     gsem[bn])
                if prefetch is None:
                    _prefetch()
                else:
                    pl.when(prefetch)(_prefetch)

                def _drain():
                    pltpu.make_async_copy(rf, out_sp.at[dblk_v.at[ch - 3]],
                                          ssem[b]).wait()
                if wait3 is None:
                    _drain()
                else:
                    pl.when(wait3)(_drain)

                for k in range(CK):
                    if k % 16 == 0:
                        w16 = wblk_v[pl.ds(ch * CK + k, 16)]
                    wk = jnp.broadcast_to(w16[k % 16], (16,))
                    for g in range(4):
                        x32 = rb[k, pl.ds(g * 32, 32)]
                        av, bv = plsc.unpack(
                            x32, format=plsc.PackFormat.INTERLEAVED,
                            preferred_element_type=jnp.float32)
                        rf[k, pl.ds(g * 32, 16)] = av * wk
                        rf[k, pl.ds(g * 32 + 16, 16)] = bv * wk
                    rf[k, pl.ds(128, 16)] = wk * onehot0

                pltpu.async_copy(rf, out_sp.at[dblk_v.at[ch]], ssem[b],
                                 add=True)

            def triple(t, _):
                step(3 * t, 0, t >= 1, None)
                step(3 * t + 1, 1, t >= 1, t < 8)
                step(3 * t + 2, 2, t >= 1, t < 8)
                return 0
            lax.fori_loop(0, BCH // 3, triple, 0)
            for b, chl in ((0, BCH - 3), (1, BCH - 2), (2, BCH - 1)):
                pltpu.make_async_copy(rowsf[b], out_sp.at[dblk_v.at[chl]],
                                      ssem[b]).wait()
            return 0
        lax.fori_loop(0, NBLK, blk_loop, 0)
        plsc.subcore_barrier()

        def wo(i, _):
            r = s * ROWS_PER_TILE + i * 64
            pltpu.sync_copy(out_sp.at[pl.ds(r, 64)],
                            out_ref.at[c, hd, pl.ds(r, 64)])
            return 0
        lax.fori_loop(0, ROWS_PER_TILE // 64, wo, 0)
        plsc.subcore_barrier()
        return 0
    lax.fori_loop(0, nheads, head_loop, 0)


def _sc_aggregate(nheads, htab, ast, adt, amx, srcf, dstf, d5):
    w4, g4 = _sc_wk(nheads, ast, adt, amx, srcf, dstf)
    g5 = g4.reshape(nheads, NTILES, NBLK, BCH, CK)
    mesh = plsc.VectorSubcoreMesh(core_axis_name="c", subcore_axis_name="s")
    kfn = pl.kernel(
        functools.partial(_sc_agg_body, nheads),
        out_type=jax.ShapeDtypeStruct((2, nheads, NPAD, F), jnp.float32),
        mesh=mesh,
        scratch_types=[
            pltpu.VMEM((CK, 128), jnp.bfloat16),
            pltpu.VMEM((CK, 128), jnp.bfloat16),
            pltpu.VMEM((CK, 128), jnp.bfloat16),
            pltpu.VMEM((CK, F), jnp.float32),
            pltpu.VMEM((CK, F), jnp.float32),
            pltpu.VMEM((CK, F), jnp.float32),
            pltpu.VMEM((BCH, CK), jnp.int32),
            pltpu.VMEM((BCH, CK), jnp.int32),
            pltpu.VMEM((BLKE,), jnp.float32),
            pltpu.SemaphoreType.DMA,
            pltpu.SemaphoreType.DMA,
            pltpu.SemaphoreType.DMA,
            pltpu.SemaphoreType.DMA,
            pltpu.SemaphoreType.DMA,
            pltpu.SemaphoreType.DMA,
            pltpu.VMEM_SHARED((NPAD, F), jnp.float32),
        ],
        compiler_params=pltpu.CompilerParams(
            needs_layout_passes=False, use_tc_tiling_on_sc=False),
    )
    return kfn(htab, g5, w4, d5)


def _l2_prep_body(p_ref, b1_ref, w2_ref, as2v_ref, ad2v_ref, pinv_ref,
                  haug_ref, as2_ref, ad2_ref, am_ref):
    i = pl.program_id(0)
    h2 = jnp.zeros((128, EMB), jnp.float32)
    for hd in range(HEADS):
        nperm = p_ref[0, hd, :, 0:128] + p_ref[1, hd, :, 0:128]
        num = jnp.dot(nperm, pinv_ref[...], preferred_element_type=jnp.float32)
        den = p_ref[0, hd, :, 128:129] + p_ref[1, hd, :, 128:129]
        g = num / (den + 1e-16) + b1_ref[hd:hd + 1, :]
        e = jnp.where(g > 0, g, jnp.exp(g) - 1.0)
        h2 = h2 + jnp.dot(e, w2_ref[hd * 128:(hd + 1) * 128, :],
                          preferred_element_type=jnp.float32)
    haug_ref[...] = h2.astype(jnp.bfloat16)
    as2_blk = jnp.sum(h2 * as2v_ref[0:1, :], axis=1, keepdims=True)
    as2_ref[0, 0, :] = as2_blk[:, 0]
    ad2_ref[0, 0, :] = jnp.sum(h2 * ad2v_ref[0:1, :], axis=1)

    ridx = i * 128 + lax.broadcasted_iota(jnp.int32, (128, 1), 0)
    masked = jnp.where(ridx < N, as2_blk, -3e38)
    cur = jnp.broadcast_to(jnp.max(masked, axis=0, keepdims=True), (16, 1))

    @pl.when(i == 0)
    def _():
        am_ref[...] = cur

    @pl.when(i > 0)
    def _():
        am_ref[...] = jnp.maximum(am_ref[...], cur)


def _l2_prep(part1, b1_2d, W2, a_src2, a_dst2, pinv):
    return pl.pallas_call(
        _l2_prep_body,
        grid=(NPAD // 128,),
        in_specs=[
            pl.BlockSpec((2, HEADS, 128, F), lambda i: (0, 0, i, 0)),
            pl.BlockSpec((HEADS, HID), lambda i: (0, 0)),
            pl.BlockSpec((HEADS * HID, EMB), lambda i: (0, 0)),
            pl.BlockSpec((1, EMB), lambda i: (0, 0)),
            pl.BlockSpec((1, EMB), lambda i: (0, 0)),
            pl.BlockSpec((128, 128), lambda i: (0, 0)),
        ],
        out_specs=[
            pl.BlockSpec((128, 128), lambda i: (i, 0)),
            pl.BlockSpec((1, 1, 128), lambda i: (i, 0, 0)),
            pl.BlockSpec((1, 1, 128), lambda i: (i, 0, 0)),
            pl.BlockSpec((16, 1), lambda i: (0, 0)),
        ],
        out_shape=[
            jax.ShapeDtypeStruct((NPAD, 128), jnp.bfloat16),
            jax.ShapeDtypeStruct((NPAD // 128, 1, 128), jnp.float32),
            jax.ShapeDtypeStruct((NPAD // 128, 1, 128), jnp.float32),
            jax.ShapeDtypeStruct((16, 1), jnp.float32),
        ],
    )(part1, b1_2d, W2, a_src2, a_dst2, pinv)


def _final_body(p_ref, b2_ref, pinv_ref, out_ref):
    nperm = p_ref[0, :, 0:128] + p_ref[1, :, 0:128]
    num = jnp.dot(nperm, pinv_ref[...], preferred_element_type=jnp.float32)
    den = p_ref[0, :, 128:129] + p_ref[1, :, 128:129]
    z = num / (den + 1e-16) + b2_ref[0:1, :]
    m = jnp.max(z, axis=1, keepdims=True)
    zs = z - m
    out_ref[...] = zs - jnp.log(jnp.sum(jnp.exp(zs), axis=1, keepdims=True))


def _final(part2, b2_2d, pinv):
    return pl.pallas_call(
        _final_body,
        grid=(79,),
        in_specs=[
            pl.BlockSpec((2, 128, F), lambda i: (0, i, 0)),
            pl.BlockSpec((1, EMB), lambda i: (0, 0)),
            pl.BlockSpec((128, 128), lambda i: (0, 0)),
        ],
        out_specs=pl.BlockSpec((128, EMB), lambda i: (i, 0)),
        out_shape=jax.ShapeDtypeStruct((N, EMB), jnp.float32),
    )(part2, b2_2d, pinv)


def kernel(x, edge_index, W1, a_src1, a_dst1, b1, W2, a_src2, a_dst2, b2):
    ei = edge_index.astype(jnp.int32)
    loop = jnp.arange(N, dtype=jnp.int32)
    padn = EPAD - (E + N)
    src = jnp.concatenate([ei[0], loop, jnp.zeros((padn,), jnp.int32)])
    dst = jnp.concatenate([ei[1], loop, jnp.full((padn,), N + 10, jnp.int32)])
    srcf = src.reshape(NTILES, NBLK, BLKE)
    dstf = dst.reshape(NTILES, NBLK, BLKE)
    d5 = dst.reshape(NTILES, NBLK, BCH, CK)

    xpad = jnp.pad(x, ((0, NPAD - N), (0, 0)))

    eye = jnp.eye(HEADS, dtype=jnp.float32)
    asbd = (eye[:, None, :] * a_src1[:, :, None]).reshape(HEADS * HID, HEADS)
    adbd = (eye[:, None, :] * a_dst1[:, :, None]).reshape(HEADS * HID, HEADS)

    fidx = jnp.arange(128)
    sidx = (fidx // 32) * 32 + (fidx % 2) * 16 + (fidx % 32) // 2
    pinv = jnp.zeros((128, 128), jnp.float32).at[sidx, fidx].set(1.0)

    haug1, as1, ad1, am1 = _l1_prep(xpad, W1, asbd, adbd)

    htab1 = haug1.reshape(HEADS * NPAD, 128)
    ast1 = as1.T.reshape(HEADS, NPAD)
    adt1 = ad1.T.reshape(HEADS, NPAD)
    amx1 = am1.T.reshape(HEADS, 16)
    part1 = _sc_aggregate(HEADS, htab1, ast1, adt1, amx1, srcf, dstf, d5)

    b1_2d = b1.reshape(HEADS, HID)
    haug2, as2, ad2, am2 = _l2_prep(part1, b1_2d, W2, a_src2, a_dst2, pinv)

    ast2 = as2.reshape(1, NPAD)
    adt2 = ad2.reshape(1, NPAD)
    amx2 = am2.T.reshape(1, 16)
    part2 = _sc_aggregate(1, haug2, ast2, adt2, amx2, srcf, dstf, d5)

    return _final(part2.reshape(2, NPAD, F), b2.reshape(1, EMB), pinv)


# speedup vs baseline: 1.4434x; 1.0646x over previous
import functools

import jax
import jax.numpy as jnp
from jax import lax
from jax.experimental import pallas as pl
from jax.experimental.pallas import tpu as pltpu
from jax.experimental.pallas import tpu_sc as plsc

N = 10000
NPAD = 10240
E = 320000
D = 128
HID = 128
HEADS = 8
EMB = 128
F = 144
NTILES = 32
NBLK = 4
BCH = 54
CK = 48
BLKE = BCH * CK
EPAD = NTILES * NBLK * BLKE
ROWS_PER_TILE = NPAD // 16
ZCH = 40


def _leaky(u):
    return jnp.maximum(u, 0.2 * u)


def _l1_prep_body(x_ref, w1_ref, asbd_ref, adbd_ref,
                  haug_ref, as_ref, ad_ref, am_ref):
    i = pl.program_id(0)
    h = jnp.dot(x_ref[...], w1_ref[...], preferred_element_type=jnp.float32)
    as_blk = jnp.dot(h, asbd_ref[...], preferred_element_type=jnp.float32)
    as_ref[...] = as_blk
    ad_ref[...] = jnp.dot(h, adbd_ref[...], preferred_element_type=jnp.float32)
    for hd in range(HEADS):
        haug_ref[hd, :, :] = h[:, hd * 128:(hd + 1) * 128].astype(jnp.bfloat16)

    ridx = i * 128 + lax.broadcasted_iota(jnp.int32, (128, HEADS), 0)
    masked = jnp.where(ridx < N, as_blk, -3e38)
    bmax = jnp.max(masked, axis=0, keepdims=True)
    cur = jnp.broadcast_to(bmax, (16, HEADS))

    @pl.when(i == 0)
    def _():
        am_ref[...] = cur

    @pl.when(i > 0)
    def _():
        am_ref[...] = jnp.maximum(am_ref[...], cur)


def _l1_prep(xpad, W1, asbd, adbd):
    return pl.pallas_call(
        _l1_prep_body,
        grid=(NPAD // 128,),
        in_specs=[
            pl.BlockSpec((128, D), lambda i: (i, 0)),
            pl.BlockSpec((D, HEADS * HID), lambda i: (0, 0)),
            pl.BlockSpec((HEADS * HID, HEADS), lambda i: (0, 0)),
            pl.BlockSpec((HEADS * HID, HEADS), lambda i: (0, 0)),
        ],
        out_specs=[
            pl.BlockSpec((HEADS, 128, 128), lambda i: (0, i, 0)),
            pl.BlockSpec((128, HEADS), lambda i: (i, 0)),
            pl.BlockSpec((128, HEADS), lambda i: (i, 0)),
            pl.BlockSpec((16, HEADS), lambda i: (0, 0)),
        ],
        out_shape=[
            jax.ShapeDtypeStruct((HEADS, NPAD, 128), jnp.bfloat16),
            jax.ShapeDtypeStruct((NPAD, HEADS), jnp.float32),
            jax.ShapeDtypeStruct((NPAD, HEADS), jnp.float32),
            jax.ShapeDtypeStruct((16, HEADS), jnp.float32),
        ],
    )(xpad, W1, asbd, adbd)


def _sc_wk_body(nheads, ast, adt, amx, srcf, dstf, wout_ref, gout_ref,
                as_v, ad_v, am_v, sblk_v, dblk_v, wv, gv):
    c = lax.axis_index("c")
    s = lax.axis_index("s")
    wid = c * 16 + s

    def head_loop(hd, _):
        pltpu.sync_copy(ast.at[hd], as_v)
        pltpu.sync_copy(adt.at[hd], ad_v)
        pltpu.sync_copy(amx.at[hd], am_v)
        mvec = am_v[...]
        base = hd * NPAD

        def blk_loop(blk, _):
            pltpu.sync_copy(srcf.at[blk, wid], sblk_v)
            pltpu.sync_copy(dstf.at[blk, wid], dblk_v)

            for g in range(BLKE // 16):
                sl = pl.ds(g * 16, 16)
                sv = sblk_v[sl]
                dv = dblk_v[sl]
                asg = plsc.load_gather(as_v, [sv])
                adg = plsc.load_gather(ad_v, [dv])
                u = asg + adg
                cc = _leaky(adg + mvec)
                wv[sl] = jnp.exp(_leaky(u) - cc)
                gv[sl] = sv + base
            pltpu.sync_copy(wv, wout_ref.at[hd, blk, wid])
            pltpu.sync_copy(gv, gout_ref.at[hd, blk, wid])
            return 0
        lax.fori_loop(0, NBLK, blk_loop, 0)
        return 0
    lax.fori_loop(0, nheads, head_loop, 0)


def _sc_wk(nheads, ast, adt, amx, srcf, dstf):
    mesh = plsc.VectorSubcoreMesh(core_axis_name="c", subcore_axis_name="s")
    kfn = pl.kernel(
        functools.partial(_sc_wk_body, nheads),
        out_type=[
            jax.ShapeDtypeStruct((nheads, NBLK, NTILES, BLKE), jnp.float32),
            jax.ShapeDtypeStruct((nheads, NBLK, NTILES, BLKE), jnp.int32),
        ],
        mesh=mesh,
        scratch_types=[
            pltpu.VMEM((NPAD,), jnp.float32),
            pltpu.VMEM((NPAD,), jnp.float32),
            pltpu.VMEM((16,), jnp.float32),
            pltpu.VMEM((BLKE,), jnp.int32),
            pltpu.VMEM((BLKE,), jnp.int32),
            pltpu.VMEM((BLKE,), jnp.float32),
            pltpu.VMEM((BLKE,), jnp.int32),
        ],
        compiler_params=pltpu.CompilerParams(
            needs_layout_passes=False, use_tc_tiling_on_sc=False),
    )
    return kfn(ast, adt, amx, srcf, dstf)


def _sc_agg_body(nheads, htab, g5, w4, d5, out_ref,
                 rb0, rb1, rb2, rf0, rf1, rf2, dblk_v, gblk_v, wblk_v,
                 gs0, gs1, gs2, ss0, ss1, ss2,
                 out_sp):
    c = lax.axis_index("c")
    s = lax.axis_index("s")
    wid = c * 16 + s
    rowsb = (rb0, rb1, rb2)
    rowsf = (rf0, rf1, rf2)
    gsem = (gs0, gs1, gs2)
    ssem = (ss0, ss1, ss2)
    zero16 = jnp.zeros((16,), jnp.float32)

    def head_loop(hd, _):
        def zr(i, _):
            for j in range(F // 16):
                rf0[i, pl.ds(j * 16, 16)] = zero16
            return 0
        lax.fori_loop(0, ZCH, zr, 0)

        def zo(i, _):
            pltpu.sync_copy(rf0.at[pl.ds(0, ZCH)],
                            out_sp.at[pl.ds(s * ROWS_PER_TILE + i * ZCH, ZCH)])
            return 0
        lax.fori_loop(0, ROWS_PER_TILE // ZCH, zo, 0)
        plsc.subcore_barrier()

        def blk_loop(blk, _):
            a1 = pltpu.async_copy(d5.at[blk, wid], dblk_v, gs0)
            a2 = pltpu.async_copy(g5.at[hd, blk, wid], gblk_v, gs1)
            a3 = pltpu.async_copy(w4.at[hd, blk, wid], wblk_v, gs2)
            a1.wait()
            a2.wait()
            a3.wait()
            pltpu.async_copy(htab.at[gblk_v.at[0]], rb0, gs0)
            pltpu.async_copy(htab.at[gblk_v.at[1]], rb1, gs1)

            def step(ch, b, wait3, prefetch):
                rb = rowsb[b]
                rf = rowsf[b]
                pltpu.make_async_copy(htab.at[gblk_v.at[ch]], rb,
                                      gsem[b]).wait()

                bn = (b + 2) % 3

                def _prefetch():
                    pltpu.async_copy(htab.at[gblk_v.at[ch + 2]], rowsb[bn],
                                     gsem[bn])
                if prefetch is None:
                    _prefetch()
                else:
                    pl.when(prefetch)(_prefetch)

                def _drain():
                    pltpu.make_async_copy(rf, out_sp.at[dblk_v.at[ch - 3]],
                                          ssem[b]).wait()
                if wait3 is None:
                    _drain()
                else:
                    pl.when(wait3)(_drain)

                col128 = jnp.full((16,), 128, jnp.int32)
                for k in range(CK):
                    if k % 16 == 0:
                        w16 = wblk_v[pl.ds(ch * CK + k, 16)]
                        kvec = lax.iota(jnp.int32, 16) + k
                        plsc.store_scatter(rf, [kvec, col128], w16)
                    wk = jnp.broadcast_to(w16[k % 16], (16,))
                    for g in range(4):
                        x32 = rb[k, pl.ds(g * 32, 32)]
                        av, bv = plsc.unpack(
                            x32, format=plsc.PackFormat.INTERLEAVED,
                            preferred_element_type=jnp.float32)
                        rf[k, pl.ds(g * 32, 16)] = av * wk
                        rf[k, pl.ds(g * 32 + 16, 16)] = bv * wk

                pltpu.async_copy(rf, out_sp.at[dblk_v.at[ch]], ssem[b],
                                 add=True)

            last = BCH // 3 - 1

            def triple(t, _):
                step(3 * t, 0, t >= 1, None)
                step(3 * t + 1, 1, t >= 1, t < last)
                step(3 * t + 2, 2, t >= 1, t < last)
                return 0
            lax.fori_loop(0, BCH // 3, triple, 0)
            for b, chl in ((0, BCH - 3), (1, BCH - 2), (2, BCH - 1)):
                pltpu.make_async_copy(rowsf[b], out_sp.at[dblk_v.at[chl]],
                                      ssem[b]).wait()
            return 0
        lax.fori_loop(0, NBLK, blk_loop, 0)
        plsc.subcore_barrier()

        def wo(i, _):
            r = s * ROWS_PER_TILE + i * 64
            pltpu.sync_copy(out_sp.at[pl.ds(r, 64)],
                            out_ref.at[c, hd, pl.ds(r, 64)])
            return 0
        lax.fori_loop(0, ROWS_PER_TILE // 64, wo, 0)
        plsc.subcore_barrier()
        return 0
    lax.fori_loop(0, nheads, head_loop, 0)


def _sc_aggregate(nheads, htab, ast, adt, amx, srcf, dstf, d5):
    w4, g4 = _sc_wk(nheads, ast, adt, amx, srcf, dstf)
    g5 = g4.reshape(nheads, NBLK, NTILES, BCH, CK)
    mesh = plsc.VectorSubcoreMesh(core_axis_name="c", subcore_axis_name="s")
    kfn = pl.kernel(
        functools.partial(_sc_agg_body, nheads),
        out_type=jax.ShapeDtypeStruct((2, nheads, NPAD, F), jnp.float32),
        mesh=mesh,
        scratch_types=[
            pltpu.VMEM((CK, 128), jnp.bfloat16),
            pltpu.VMEM((CK, 128), jnp.bfloat16),
            pltpu.VMEM((CK, 128), jnp.bfloat16),
            pltpu.VMEM((CK, F), jnp.float32),
            pltpu.VMEM((CK, F), jnp.float32),
            pltpu.VMEM((CK, F), jnp.float32),
            pltpu.VMEM((BCH, CK), jnp.int32),
            pltpu.VMEM((BCH, CK), jnp.int32),
            pltpu.VMEM((BLKE,), jnp.float32),
            pltpu.SemaphoreType.DMA,
            pltpu.SemaphoreType.DMA,
            pltpu.SemaphoreType.DMA,
            pltpu.SemaphoreType.DMA,
            pltpu.SemaphoreType.DMA,
            pltpu.SemaphoreType.DMA,
            pltpu.VMEM_SHARED((NPAD, F), jnp.float32),
        ],
        compiler_params=pltpu.CompilerParams(
            needs_layout_passes=False, use_tc_tiling_on_sc=False),
    )
    return kfn(htab, g5, w4, d5)


def _l2_prep_body(p_ref, b1_ref, w2_ref, as2v_ref, ad2v_ref, pinv_ref,
                  haug_ref, as2_ref, ad2_ref, am_ref):
    i = pl.program_id(0)
    h2 = jnp.zeros((128, EMB), jnp.float32)
    for hd in range(HEADS):
        nperm = p_ref[0, hd, :, 0:128] + p_ref[1, hd, :, 0:128]
        num = jnp.dot(nperm, pinv_ref[...], preferred_element_type=jnp.float32)
        den = p_ref[0, hd, :, 128:129] + p_ref[1, hd, :, 128:129]
        g = num / (den + 1e-16) + b1_ref[hd:hd + 1, :]
        e = jnp.where(g > 0, g, jnp.exp(g) - 1.0)
        h2 = h2 + jnp.dot(e, w2_ref[hd * 128:(hd + 1) * 128, :],
                          preferred_element_type=jnp.float32)
    haug_ref[...] = h2.astype(jnp.bfloat16)
    as2_blk = jnp.sum(h2 * as2v_ref[0:1, :], axis=1, keepdims=True)
    as2_ref[0, 0, :] = as2_blk[:, 0]
    ad2_ref[0, 0, :] = jnp.sum(h2 * ad2v_ref[0:1, :], axis=1)

    ridx = i * 128 + lax.broadcasted_iota(jnp.int32, (128, 1), 0)
    masked = jnp.where(ridx < N, as2_blk, -3e38)
    cur = jnp.broadcast_to(jnp.max(masked, axis=0, keepdims=True), (16, 1))

    @pl.when(i == 0)
    def _():
        am_ref[...] = cur

    @pl.when(i > 0)
    def _():
        am_ref[...] = jnp.maximum(am_ref[...], cur)


def _l2_prep(part1, b1_2d, W2, a_src2, a_dst2, pinv):
    return pl.pallas_call(
        _l2_prep_body,
        grid=(NPAD // 128,),
        in_specs=[
            pl.BlockSpec((2, HEADS, 128, F), lambda i: (0, 0, i, 0)),
            pl.BlockSpec((HEADS, HID), lambda i: (0, 0)),
            pl.BlockSpec((HEADS * HID, EMB), lambda i: (0, 0)),
            pl.BlockSpec((1, EMB), lambda i: (0, 0)),
            pl.BlockSpec((1, EMB), lambda i: (0, 0)),
            pl.BlockSpec((128, 128), lambda i: (0, 0)),
        ],
        out_specs=[
            pl.BlockSpec((128, 128), lambda i: (i, 0)),
            pl.BlockSpec((1, 1, 128), lambda i: (i, 0, 0)),
            pl.BlockSpec((1, 1, 128), lambda i: (i, 0, 0)),
            pl.BlockSpec((16, 1), lambda i: (0, 0)),
        ],
        out_shape=[
            jax.ShapeDtypeStruct((NPAD, 128), jnp.bfloat16),
            jax.ShapeDtypeStruct((NPAD // 128, 1, 128), jnp.float32),
            jax.ShapeDtypeStruct((NPAD // 128, 1, 128), jnp.float32),
            jax.ShapeDtypeStruct((16, 1), jnp.float32),
        ],
    )(part1, b1_2d, W2, a_src2, a_dst2, pinv)


def _final_body(p_ref, b2_ref, pinv_ref, out_ref):
    nperm = p_ref[0, :, 0:128] + p_ref[1, :, 0:128]
    num = jnp.dot(nperm, pinv_ref[...], preferred_element_type=jnp.float32)
    den = p_ref[0, :, 128:129] + p_ref[1, :, 128:129]
    z = num / (den + 1e-16) + b2_ref[0:1, :]
    m = jnp.max(z, axis=1, keepdims=True)
    zs = z - m
    out_ref[...] = zs - jnp.log(jnp.sum(jnp.exp(zs), axis=1, keepdims=True))


def _final(part2, b2_2d, pinv):
    return pl.pallas_call(
        _final_body,
        grid=(79,),
        in_specs=[
            pl.BlockSpec((2, 128, F), lambda i: (0, i, 0)),
            pl.BlockSpec((1, EMB), lambda i: (0, 0)),
            pl.BlockSpec((128, 128), lambda i: (0, 0)),
        ],
        out_specs=pl.BlockSpec((128, EMB), lambda i: (i, 0)),
        out_shape=jax.ShapeDtypeStruct((N, EMB), jnp.float32),
    )(part2, b2_2d, pinv)


def kernel(x, edge_index, W1, a_src1, a_dst1, b1, W2, a_src2, a_dst2, b2):
    ei = edge_index.astype(jnp.int32)
    loop = jnp.arange(N, dtype=jnp.int32)
    padn = EPAD - (E + N)
    src = jnp.concatenate([ei[0], loop, jnp.zeros((padn,), jnp.int32)])
    dst = jnp.concatenate([ei[1], loop, jnp.full((padn,), N + 10, jnp.int32)])
    srcf = src.reshape(NBLK, NTILES, BLKE)
    dstf = dst.reshape(NBLK, NTILES, BLKE)
    d5 = dst.reshape(NBLK, NTILES, BCH, CK)

    xpad = jnp.pad(x, ((0, NPAD - N), (0, 0)))

    eye = jnp.eye(HEADS, dtype=jnp.float32)
    asbd = (eye[:, None, :] * a_src1[:, :, None]).reshape(HEADS * HID, HEADS)
    adbd = (eye[:, None, :] * a_dst1[:, :, None]).reshape(HEADS * HID, HEADS)

    fidx = jnp.arange(128)
    sidx = (fidx // 32) * 32 + (fidx % 2) * 16 + (fidx % 32) // 2
    pinv = jnp.zeros((128, 128), jnp.float32).at[sidx, fidx].set(1.0)

    haug1, as1, ad1, am1 = _l1_prep(xpad, W1, asbd, adbd)

    htab1 = haug1.reshape(HEADS * NPAD, 128)
    ast1 = as1.T.reshape(HEADS, NPAD)
    adt1 = ad1.T.reshape(HEADS, NPAD)
    amx1 = am1.T.reshape(HEADS, 16)
    part1 = _sc_aggregate(HEADS, htab1, ast1, adt1, amx1, srcf, dstf, d5)

    b1_2d = b1.reshape(HEADS, HID)
    haug2, as2, ad2, am2 = _l2_prep(part1, b1_2d, W2, a_src2, a_dst2, pinv)

    ast2 = as2.reshape(1, NPAD)
    adt2 = ad2.reshape(1, NPAD)
    amx2 = am2.T.reshape(1, 16)
    part2 = _sc_aggregate(1, haug2, ast2, adt2, amx2, srcf, dstf, d5)

    return _final(part2.reshape(2, NPAD, F), b2.reshape(1, EMB), pinv)


# speedup vs baseline: 1.4666x; 1.0161x over previous
import functools

import jax
import jax.numpy as jnp
from jax import lax
from jax.experimental import pallas as pl
from jax.experimental.pallas import tpu as pltpu
from jax.experimental.pallas import tpu_sc as plsc

N = 10000
NPAD = 10240
E = 320000
D = 128
HID = 128
HEADS = 8
EMB = 128
F = 144
NTILES = 32
NBLK = 4
BCH = 54
CK = 48
BLKE = BCH * CK
EPAD = NTILES * NBLK * BLKE
ROWS_PER_TILE = NPAD // 16
ZCH = 40


def _leaky(u):
    return jnp.maximum(u, 0.2 * u)


def _l1_prep_body(x_ref, w1_ref, asbd_ref, adbd_ref,
                  haug_ref, as_ref, ad_ref, am_ref):
    i = pl.program_id(0)
    h = jnp.dot(x_ref[...], w1_ref[...], preferred_element_type=jnp.float32)
    as_blk = jnp.dot(h, asbd_ref[...], preferred_element_type=jnp.float32)
    as_ref[...] = as_blk
    ad_ref[...] = jnp.dot(h, adbd_ref[...], preferred_element_type=jnp.float32)
    for hd in range(HEADS):
        haug_ref[hd, :, :] = h[:, hd * 128:(hd + 1) * 128].astype(jnp.bfloat16)

    ridx = i * 128 + lax.broadcasted_iota(jnp.int32, (128, HEADS), 0)
    masked = jnp.where(ridx < N, as_blk, -3e38)
    bmax = jnp.max(masked, axis=0, keepdims=True)
    cur = jnp.broadcast_to(bmax, (16, HEADS))

    @pl.when(i == 0)
    def _():
        am_ref[...] = cur

    @pl.when(i > 0)
    def _():
        am_ref[...] = jnp.maximum(am_ref[...], cur)


def _l1_prep(xpad, W1, asbd, adbd):
    return pl.pallas_call(
        _l1_prep_body,
        grid=(NPAD // 128,),
        in_specs=[
            pl.BlockSpec((128, D), lambda i: (i, 0)),
            pl.BlockSpec((D, HEADS * HID), lambda i: (0, 0)),
            pl.BlockSpec((HEADS * HID, HEADS), lambda i: (0, 0)),
            pl.BlockSpec((HEADS * HID, HEADS), lambda i: (0, 0)),
        ],
        out_specs=[
            pl.BlockSpec((HEADS, 128, 128), lambda i: (0, i, 0)),
            pl.BlockSpec((128, HEADS), lambda i: (i, 0)),
            pl.BlockSpec((128, HEADS), lambda i: (i, 0)),
            pl.BlockSpec((16, HEADS), lambda i: (0, 0)),
        ],
        out_shape=[
            jax.ShapeDtypeStruct((HEADS, NPAD, 128), jnp.bfloat16),
            jax.ShapeDtypeStruct((NPAD, HEADS), jnp.float32),
            jax.ShapeDtypeStruct((NPAD, HEADS), jnp.float32),
            jax.ShapeDtypeStruct((16, HEADS), jnp.float32),
        ],
    )(xpad, W1, asbd, adbd)


def _sc_wk_body(nheads, ast, adt, amx, srcf, dstf, wout_ref, gout_ref,
                as_v, ad_v, am_v, sblk_v, dblk_v, wv, gv, se0, se1, se2):
    c = lax.axis_index("c")
    s = lax.axis_index("s")
    wid = c * 16 + s

    def head_loop(hd, _):
        t1 = pltpu.async_copy(ast.at[hd], as_v, se0)
        t2 = pltpu.async_copy(adt.at[hd], ad_v, se1)
        t3 = pltpu.async_copy(amx.at[hd], am_v, se2)
        t1.wait()
        t2.wait()
        t3.wait()
        mvec = am_v[...]
        base = hd * NPAD

        def blk_loop(blk, _):
            l1 = pltpu.async_copy(srcf.at[blk, wid], sblk_v, se0)
            l2 = pltpu.async_copy(dstf.at[blk, wid], dblk_v, se1)
            l1.wait()
            l2.wait()

            for g in range(BLKE // 16):
                sl = pl.ds(g * 16, 16)
                sv = sblk_v[sl]
                dv = dblk_v[sl]
                asg = plsc.load_gather(as_v, [sv])
                adg = plsc.load_gather(ad_v, [dv])
                u = asg + adg
                cc = _leaky(adg + mvec)
                wv[sl] = jnp.exp(_leaky(u) - cc)
                gv[sl] = sv + base
            w1a = pltpu.async_copy(wv, wout_ref.at[hd, blk, wid], se0)
            w2a = pltpu.async_copy(gv, gout_ref.at[hd, blk, wid], se1)
            w1a.wait()
            w2a.wait()
            return 0
        lax.fori_loop(0, NBLK, blk_loop, 0)
        return 0
    lax.fori_loop(0, nheads, head_loop, 0)


def _sc_wk(nheads, ast, adt, amx, srcf, dstf):
    mesh = plsc.VectorSubcoreMesh(core_axis_name="c", subcore_axis_name="s")
    kfn = pl.kernel(
        functools.partial(_sc_wk_body, nheads),
        out_type=[
            jax.ShapeDtypeStruct((nheads, NBLK, NTILES, BLKE), jnp.float32),
            jax.ShapeDtypeStruct((nheads, NBLK, NTILES, BLKE), jnp.int32),
        ],
        mesh=mesh,
        scratch_types=[
            pltpu.VMEM((NPAD,), jnp.float32),
            pltpu.VMEM((NPAD,), jnp.float32),
            pltpu.VMEM((16,), jnp.float32),
            pltpu.VMEM((BLKE,), jnp.int32),
            pltpu.VMEM((BLKE,), jnp.int32),
            pltpu.VMEM((BLKE,), jnp.float32),
            pltpu.VMEM((BLKE,), jnp.int32),
            pltpu.SemaphoreType.DMA,
            pltpu.SemaphoreType.DMA,
            pltpu.SemaphoreType.DMA,
        ],
        compiler_params=pltpu.CompilerParams(
            needs_layout_passes=False, use_tc_tiling_on_sc=False),
    )
    return kfn(ast, adt, amx, srcf, dstf)


def _sc_agg_body(nheads, htab, g5, w4, d5, out_ref,
                 rb0, rb1, rb2, rf0, rf1, rf2, dblk_v, gblk_v, wblk_v,
                 gs0, gs1, gs2, ss0, ss1, ss2,
                 out_sp):
    c = lax.axis_index("c")
    s = lax.axis_index("s")
    wid = c * 16 + s
    rowsb = (rb0, rb1, rb2)
    rowsf = (rf0, rf1, rf2)
    gsem = (gs0, gs1, gs2)
    ssem = (ss0, ss1, ss2)
    zero16 = jnp.zeros((16,), jnp.float32)

    def head_loop(hd, _):
        def zr(i, _):
            for j in range(F // 16):
                rf0[i, pl.ds(j * 16, 16)] = zero16
            return 0
        lax.fori_loop(0, ZCH, zr, 0)

        def zo(i, _):
            pltpu.sync_copy(rf0.at[pl.ds(0, ZCH)],
                            out_sp.at[pl.ds(s * ROWS_PER_TILE + i * ZCH, ZCH)])
            return 0
        lax.fori_loop(0, ROWS_PER_TILE // ZCH, zo, 0)
        plsc.subcore_barrier()

        def blk_loop(blk, _):
            a1 = pltpu.async_copy(d5.at[blk, wid], dblk_v, gs0)
            a2 = pltpu.async_copy(g5.at[hd, blk, wid], gblk_v, gs1)
            a3 = pltpu.async_copy(w4.at[hd, blk, wid], wblk_v, gs2)
            a1.wait()
            a2.wait()
            a3.wait()
            pltpu.async_copy(htab.at[gblk_v.at[0]], rb0, gs0)
            pltpu.async_copy(htab.at[gblk_v.at[1]], rb1, gs1)

            def step(ch, b, wait3, prefetch):
                rb = rowsb[b]
                rf = rowsf[b]
                pltpu.make_async_copy(htab.at[gblk_v.at[ch]], rb,
                                      gsem[b]).wait()

                bn = (b + 2) % 3

                def _prefetch():
                    pltpu.async_copy(htab.at[gblk_v.at[ch + 2]], rowsb[bn],
                                     gsem[bn])
                if prefetch is None:
                    _prefetch()
                else:
                    pl.when(prefetch)(_prefetch)

                def _drain():
                    pltpu.make_async_copy(rf, out_sp.at[dblk_v.at[ch - 3]],
                                          ssem[b]).wait()
                if wait3 is None:
                    _drain()
                else:
                    pl.when(wait3)(_drain)

                col128 = jnp.full((16,), 128, jnp.int32)
                for k in range(CK):
                    if k % 16 == 0:
                        w16 = wblk_v[pl.ds(ch * CK + k, 16)]
                        kvec = lax.iota(jnp.int32, 16) + k
                        plsc.store_scatter(rf, [kvec, col128], w16)
                    wk = jnp.broadcast_to(w16[k % 16], (16,))
                    for g in range(4):
                        x32 = rb[k, pl.ds(g * 32, 32)]
                        av, bv = plsc.unpack(
                            x32, format=plsc.PackFormat.INTERLEAVED,
                            preferred_element_type=jnp.float32)
                        rf[k, pl.ds(g * 32, 16)] = av * wk
                        rf[k, pl.ds(g * 32 + 16, 16)] = bv * wk

                pltpu.async_copy(rf, out_sp.at[dblk_v.at[ch]], ssem[b],
                                 add=True)

            last = BCH // 3 - 1

            def triple(t, _):
                step(3 * t, 0, t >= 1, None)
                step(3 * t + 1, 1, t >= 1, t < last)
                step(3 * t + 2, 2, t >= 1, t < last)
                return 0
            lax.fori_loop(0, BCH // 3, triple, 0)
            for b, chl in ((0, BCH - 3), (1, BCH - 2), (2, BCH - 1)):
                pltpu.make_async_copy(rowsf[b], out_sp.at[dblk_v.at[chl]],
                                      ssem[b]).wait()
            return 0
        lax.fori_loop(0, NBLK, blk_loop, 0)
        plsc.subcore_barrier()

        def wo(i, _):
            r = s * ROWS_PER_TILE + i * 64
            pltpu.sync_copy(out_sp.at[pl.ds(r, 64)],
                            out_ref.at[c, hd, pl.ds(r, 64)])
            return 0
        lax.fori_loop(0, ROWS_PER_TILE // 64, wo, 0)
        plsc.subcore_barrier()
        return 0
    lax.fori_loop(0, nheads, head_loop, 0)


def _sc_aggregate(nheads, htab, ast, adt, amx, srcf, dstf, d5):
    w4, g4 = _sc_wk(nheads, ast, adt, amx, srcf, dstf)
    g5 = g4.reshape(nheads, NBLK, NTILES, BCH, CK)
    mesh = plsc.VectorSubcoreMesh(core_axis_name="c", subcore_axis_name="s")
    kfn = pl.kernel(
        functools.partial(_sc_agg_body, nheads),
        out_type=jax.ShapeDtypeStruct((2, nheads, NPAD, F), jnp.float32),
        mesh=mesh,
        scratch_types=[
            pltpu.VMEM((CK, 128), jnp.bfloat16),
            pltpu.VMEM((CK, 128), jnp.bfloat16),
            pltpu.VMEM((CK, 128), jnp.bfloat16),
            pltpu.VMEM((CK, F), jnp.float32),
            pltpu.VMEM((CK, F), jnp.float32),
            pltpu.VMEM((CK, F), jnp.float32),
            pltpu.VMEM((BCH, CK), jnp.int32),
            pltpu.VMEM((BCH, CK), jnp.int32),
            pltpu.VMEM((BLKE,), jnp.float32),
            pltpu.SemaphoreType.DMA,
            pltpu.SemaphoreType.DMA,
            pltpu.SemaphoreType.DMA,
            pltpu.SemaphoreType.DMA,
            pltpu.SemaphoreType.DMA,
            pltpu.SemaphoreType.DMA,
            pltpu.VMEM_SHARED((NPAD, F), jnp.float32),
        ],
        compiler_params=pltpu.CompilerParams(
            needs_layout_passes=False, use_tc_tiling_on_sc=False),
    )
    return kfn(htab, g5, w4, d5)


def _l2_prep_body(p_ref, b1_ref, w2_ref, as2v_ref, ad2v_ref, pinv_ref,
                  haug_ref, as2_ref, ad2_ref, am_ref):
    i = pl.program_id(0)
    h2 = jnp.zeros((128, EMB), jnp.float32)
    for hd in range(HEADS):
        nperm = p_ref[0, hd, :, 0:128] + p_ref[1, hd, :, 0:128]
        num = jnp.dot(nperm, pinv_ref[...], preferred_element_type=jnp.float32)
        den = p_ref[0, hd, :, 128:129] + p_ref[1, hd, :, 128:129]
        g = num / (den + 1e-16) + b1_ref[hd:hd + 1, :]
        e = jnp.where(g > 0, g, jnp.exp(g) - 1.0)
        h2 = h2 + jnp.dot(e, w2_ref[hd * 128:(hd + 1) * 128, :],
                          preferred_element_type=jnp.float32)
    haug_ref[...] = h2.astype(jnp.bfloat16)
    as2_blk = jnp.sum(h2 * as2v_ref[0:1, :], axis=1, keepdims=True)
    as2_ref[0, 0, :] = as2_blk[:, 0]
    ad2_ref[0, 0, :] = jnp.sum(h2 * ad2v_ref[0:1, :], axis=1)

    ridx = i * 128 + lax.broadcasted_iota(jnp.int32, (128, 1), 0)
    masked = jnp.where(ridx < N, as2_blk, -3e38)
    cur = jnp.broadcast_to(jnp.max(masked, axis=0, keepdims=True), (16, 1))

    @pl.when(i == 0)
    def _():
        am_ref[...] = cur

    @pl.when(i > 0)
    def _():
        am_ref[...] = jnp.maximum(am_ref[...], cur)


def _l2_prep(part1, b1_2d, W2, a_src2, a_dst2, pinv):
    return pl.pallas_call(
        _l2_prep_body,
        grid=(NPAD // 128,),
        in_specs=[
            pl.BlockSpec((2, HEADS, 128, F), lambda i: (0, 0, i, 0)),
            pl.BlockSpec((HEADS, HID), lambda i: (0, 0)),
            pl.BlockSpec((HEADS * HID, EMB), lambda i: (0, 0)),
            pl.BlockSpec((1, EMB), lambda i: (0, 0)),
            pl.BlockSpec((1, EMB), lambda i: (0, 0)),
            pl.BlockSpec((128, 128), lambda i: (0, 0)),
        ],
        out_specs=[
            pl.BlockSpec((128, 128), lambda i: (i, 0)),
            pl.BlockSpec((1, 1, 128), lambda i: (i, 0, 0)),
            pl.BlockSpec((1, 1, 128), lambda i: (i, 0, 0)),
            pl.BlockSpec((16, 1), lambda i: (0, 0)),
        ],
        out_shape=[
            jax.ShapeDtypeStruct((NPAD, 128), jnp.bfloat16),
            jax.ShapeDtypeStruct((NPAD // 128, 1, 128), jnp.float32),
            jax.ShapeDtypeStruct((NPAD // 128, 1, 128), jnp.float32),
            jax.ShapeDtypeStruct((16, 1), jnp.float32),
        ],
    )(part1, b1_2d, W2, a_src2, a_dst2, pinv)


def _final_body(p_ref, b2_ref, pinv_ref, out_ref):
    nperm = p_ref[0, :, 0:128] + p_ref[1, :, 0:128]
    num = jnp.dot(nperm, pinv_ref[...], preferred_element_type=jnp.float32)
    den = p_ref[0, :, 128:129] + p_ref[1, :, 128:129]
    z = num / (den + 1e-16) + b2_ref[0:1, :]
    m = jnp.max(z, axis=1, keepdims=True)
    zs = z - m
    out_ref[...] = zs - jnp.log(jnp.sum(jnp.exp(zs), axis=1, keepdims=True))


def _final(part2, b2_2d, pinv):
    return pl.pallas_call(
        _final_body,
        grid=(79,),
        in_specs=[
            pl.BlockSpec((2, 128, F), lambda i: (0, i, 0)),
            pl.BlockSpec((1, EMB), lambda i: (0, 0)),
            pl.BlockSpec((128, 128), lambda i: (0, 0)),
        ],
        out_specs=pl.BlockSpec((128, EMB), lambda i: (i, 0)),
        out_shape=jax.ShapeDtypeStruct((N, EMB), jnp.float32),
    )(part2, b2_2d, pinv)


def kernel(x, edge_index, W1, a_src1, a_dst1, b1, W2, a_src2, a_dst2, b2):
    ei = edge_index.astype(jnp.int32)
    loop = jnp.arange(N, dtype=jnp.int32)
    padn = EPAD - (E + N)
    src = jnp.concatenate([ei[0], loop, jnp.zeros((padn,), jnp.int32)])
    dst = jnp.concatenate([ei[1], loop, jnp.full((padn,), N + 10, jnp.int32)])
    srcf = src.reshape(NBLK, NTILES, BLKE)
    dstf = dst.reshape(NBLK, NTILES, BLKE)
    d5 = dst.reshape(NBLK, NTILES, BCH, CK)

    xpad = jnp.pad(x, ((0, NPAD - N), (0, 0)))

    eye = jnp.eye(HEADS, dtype=jnp.float32)
    asbd = (eye[:, None, :] * a_src1[:, :, None]).reshape(HEADS * HID, HEADS)
    adbd = (eye[:, None, :] * a_dst1[:, :, None]).reshape(HEADS * HID, HEADS)

    fidx = jnp.arange(128)
    sidx = (fidx // 32) * 32 + (fidx % 2) * 16 + (fidx % 32) // 2
    pinv = jnp.zeros((128, 128), jnp.float32).at[sidx, fidx].set(1.0)

    haug1, as1, ad1, am1 = _l1_prep(xpad, W1, asbd, adbd)

    htab1 = haug1.reshape(HEADS * NPAD, 128)
    ast1 = as1.T.reshape(HEADS, NPAD)
    adt1 = ad1.T.reshape(HEADS, NPAD)
    amx1 = am1.T.reshape(HEADS, 16)
    part1 = _sc_aggregate(HEADS, htab1, ast1, adt1, amx1, srcf, dstf, d5)

    b1_2d = b1.reshape(HEADS, HID)
    haug2, as2, ad2, am2 = _l2_prep(part1, b1_2d, W2, a_src2, a_dst2, pinv)

    ast2 = as2.reshape(1, NPAD)
    adt2 = ad2.reshape(1, NPAD)
    amx2 = am2.T.reshape(1, 16)
    part2 = _sc_aggregate(1, haug2, ast2, adt2, amx2, srcf, dstf, d5)

    return _final(part2.reshape(2, NPAD, F), b2.reshape(1, EMB), pinv)
